# Initial kernel scaffold; baseline (speedup 1.0000x reference)
import jax
import jax.numpy as jnp
from jax.experimental import pallas as pl


def kernel(x, edge_index, edges, conv1_W, conv1_b, conv2_W, conv2_b, bn_gamma, bn_beta, bn_mean, bn_var, link_W, link_b, type_W1, type_b1, type_W2, type_b2, mod_W1, mod_b1, mod_W2, mod_b2):
    raise NotImplementedError("write your pallas kernel here")


# speedup vs baseline: 7.6361x; 7.6361x over previous
import functools

import jax
import jax.numpy as jnp
from jax import lax
from jax.experimental import pallas as pl
from jax.experimental.pallas import tpu as pltpu
from jax.experimental.pallas import tpu_sc as plsc

N_NODES = 100000
N_EDGES = 1600000
N_PAIRS = 1000000

NC = 2
NS = 16
NW = NC * NS

CH = 128

ACC = 102400
HALF = ACC // 2
DEGR = HALF + CH
ZR = DEGR // NS
OR_ = HALF // NS
EROWS = 12544
EPAD = EROWS * CH
ETA = EROWS // NS
STG = ETA // 2
PROWS = 7936
PPAD = PROWS * CH
PT = PROWS // NW

_mesh = functools.partial(
    plsc.VectorSubcoreMesh, core_axis_name="c", subcore_axis_name="s",
    num_cores=NC, num_subcores=NS)

_SC_PARAMS = pltpu.CompilerParams(use_tc_tiling_on_sc=False)


def _f32(*shape):
  return jax.ShapeDtypeStruct(shape, jnp.float32)


def _remap(dvm, ibuf, j, base):
  for kk in range(CH // 16):
    v = dvm[j, pl.ds(kk * 16, 16)]
    local = v - base
    ok = (v >= base) & (local < HALF)
    ibuf[pl.ds(kk * 16, 16)] = jnp.where(ok, local, HALF + (v & (CH - 1)))


def _sc_deg(dst2d, zeros8, ones8):
  @functools.partial(
      pl.kernel, mesh=_mesh(), compiler_params=_SC_PARAMS,
      out_type=_f32(ACC, 8),
      scratch_types=[
          pltpu.VMEM((STG, CH), jnp.int32),
          pltpu.VMEM((CH,), jnp.int32),
          pltpu.VMEM((CH, 8), jnp.float32),
          pltpu.VMEM_SHARED((DEGR, 8), jnp.float32),
      ],
  )
  def k(dst_hbm, z_hbm, o_hbm, out_hbm, dvm, ibuf, ovm, acc):
    c = lax.axis_index("c")
    s = lax.axis_index("s")
    base = c * HALF
    pltpu.sync_copy(z_hbm.at[pl.ds(0, ZR)], acc.at[pl.ds(s * ZR, ZR)])
    pltpu.sync_copy(o_hbm, ovm)
    plsc.subcore_barrier()

    def stage(t, _):
      pltpu.sync_copy(dst_hbm.at[pl.ds(s * ETA + t * STG, STG)], dvm)

      def body(j, _):
        _remap(dvm, ibuf, j, base)
        pltpu.sync_copy(ovm, acc.at[ibuf], add=True)
        return ()

      lax.fori_loop(0, STG, body, (), unroll=False)
      return ()

    lax.fori_loop(0, ETA // STG, stage, (), unroll=False)
    plsc.subcore_barrier()
    pltpu.sync_copy(acc.at[pl.ds(s * OR_, OR_)],
                    out_hbm.at[pl.ds(base + s * OR_, OR_)])

  return k(dst2d, zeros8, ones8)


def _sc_segsum(src2d, dst2d, tabs, zeros8):
  @functools.partial(
      pl.kernel, mesh=_mesh(), compiler_params=_SC_PARAMS,
      out_type=tuple(_f32(ACC, 8) for _ in tabs),
      scratch_types=[
          pltpu.VMEM((STG, CH), jnp.int32),
          pltpu.VMEM((STG, CH), jnp.int32),
          pltpu.VMEM((CH,), jnp.int32),
          pltpu.VMEM((CH, 8), jnp.float32),
          pltpu.VMEM_SHARED((DEGR, 8), jnp.float32),
      ],
  )
  def k(src_hbm, dst_hbm, *rest):
    tab_hbms = rest[:len(tabs)]
    z_hbm = rest[len(tabs)]
    out_hbms = rest[len(tabs) + 1:len(tabs) + 1 + len(tabs)]
    svm, dvm, ibuf, gbuf, acc = rest[len(tabs) + 1 + len(tabs):]
    c = lax.axis_index("c")
    s = lax.axis_index("s")
    base = c * HALF

    for tab_hbm, out_hbm in zip(tab_hbms, out_hbms):
      pltpu.sync_copy(z_hbm.at[pl.ds(0, ZR)], acc.at[pl.ds(s * ZR, ZR)])
      plsc.subcore_barrier()

      def stage(t, _):
        pltpu.sync_copy(src_hbm.at[pl.ds(s * ETA + t * STG, STG)], svm)
        pltpu.sync_copy(dst_hbm.at[pl.ds(s * ETA + t * STG, STG)], dvm)

        def body(j, _):
          _remap(dvm, ibuf, j, base)
          pltpu.sync_copy(tab_hbm.at[svm.at[j]], gbuf)
          pltpu.sync_copy(gbuf, acc.at[ibuf], add=True)
          return ()

        lax.fori_loop(0, STG, body, (), unroll=False)
        return ()

      lax.fori_loop(0, ETA // STG, stage, (), unroll=False)
      plsc.subcore_barrier()
      pltpu.sync_copy(acc.at[pl.ds(s * OR_, OR_)],
                      out_hbm.at[pl.ds(base + s * OR_, OR_)])
      plsc.subcore_barrier()

  return k(src2d, dst2d, *tabs, zeros8)


def _sc_pair_gather(idx0, idx1, tab):
  @functools.partial(
      pl.kernel, mesh=_mesh(), compiler_params=_SC_PARAMS,
      out_type=(_f32(PPAD, 16), _f32(PPAD, 16)),
      scratch_types=[
          pltpu.VMEM((PT, CH), jnp.int32),
          pltpu.VMEM((PT, CH), jnp.int32),
          pltpu.VMEM((CH, 16), jnp.float32),
          pltpu.VMEM((CH, 16), jnp.float32),
      ],
  )
  def k(i0_hbm, i1_hbm, tab_hbm, n0_hbm, n1_hbm, i0vm, i1vm, b0, b1):
    c = lax.axis_index("c")
    s = lax.axis_index("s")
    w = c * NS + s
    pltpu.sync_copy(i0_hbm.at[pl.ds(w * PT, PT)], i0vm)
    pltpu.sync_copy(i1_hbm.at[pl.ds(w * PT, PT)], i1vm)

    def body(j, _):
      base = (w * PT + j) * CH
      pltpu.sync_copy(tab_hbm.at[i0vm.at[j]], b0)
      pltpu.sync_copy(b0, n0_hbm.at[pl.ds(base, CH)])
      pltpu.sync_copy(tab_hbm.at[i1vm.at[j]], b1)
      pltpu.sync_copy(b1, n1_hbm.at[pl.ds(base, CH)])
      return ()

    lax.fori_loop(0, PT, body, (), unroll=False)

  return k(idx0, idx1, tab)


_TB = 2048
_PB = 4096


def _dinv8(deg8):
  return 1.0 / jnp.sqrt(1.0 + deg8)


def _tc_xs(xpad, deg8):
  def body(x_ref, d_ref, o_ref):
    o_ref[...] = x_ref[...] * _dinv8(d_ref[...])

  return pl.pallas_call(
      body,
      grid=(ACC // _TB,),
      in_specs=[
          pl.BlockSpec((_TB, 8), lambda i: (i, 0)),
          pl.BlockSpec((_TB, 8), lambda i: (i, 0)),
      ],
      out_specs=pl.BlockSpec((_TB, 8), lambda i: (i, 0)),
      out_shape=_f32(ACC, 8),
  )(xpad, deg8)


def _tc_g(a1, xs, deg8, W1, b1, W2f):
  def body(a_ref, x_ref, d_ref, w1_ref, b1_ref, w2_ref, ol_ref, or_ref):
    dinv = _dinv8(d_ref[...])
    pre1 = dinv * (a_ref[...] + x_ref[...])
    h1 = jax.nn.relu(
        jnp.dot(pre1, w1_ref[...], preferred_element_type=jnp.float32)
        + b1_ref[...])
    g = jnp.dot(h1, w2_ref[...], preferred_element_type=jnp.float32)
    g = jnp.broadcast_to(dinv[:, :1], g.shape) * g
    ol_ref[...] = g[:, :8]
    or_ref[...] = g[:, 8:]

  return pl.pallas_call(
      body,
      grid=(ACC // _TB,),
      in_specs=[
          pl.BlockSpec((_TB, 8), lambda i: (i, 0)),
          pl.BlockSpec((_TB, 8), lambda i: (i, 0)),
          pl.BlockSpec((_TB, 8), lambda i: (i, 0)),
          pl.BlockSpec((8, 32), lambda i: (0, 0)),
          pl.BlockSpec((1, 32), lambda i: (0, 0)),
          pl.BlockSpec((32, 16), lambda i: (0, 0)),
      ],
      out_specs=[
          pl.BlockSpec((_TB, 8), lambda i: (i, 0)),
          pl.BlockSpec((_TB, 8), lambda i: (i, 0)),
      ],
      out_shape=[_f32(ACC, 8), _f32(ACC, 8)],
  )(a1, xs, deg8, W1, b1, W2f)


def _tc_h2(a2l, a2r, gl, gr, deg8, b2f):
  def body(al_ref, ar_ref, gl_ref, gr_ref, d_ref, b_ref, o_ref):
    dinv = _dinv8(d_ref[...])
    hl = dinv * (al_ref[...] + gl_ref[...])
    hr = dinv * (ar_ref[...] + gr_ref[...])
    o_ref[...] = jnp.concatenate([hl, hr], axis=1) + b_ref[...]

  blk8 = pl.BlockSpec((_TB, 8), lambda i: (i, 0))
  return pl.pallas_call(
      body,
      grid=(ACC // _TB,),
      in_specs=[blk8, blk8, blk8, blk8, blk8,
                pl.BlockSpec((1, 16), lambda i: (0, 0))],
      out_specs=pl.BlockSpec((_TB, 16), lambda i: (i, 0)),
      out_shape=_f32(ACC, 16),
  )(a2l, a2r, gl, gr, deg8, b2f)


def _softmax(z):
  z = z - jnp.max(z, axis=-1, keepdims=True)
  e = jnp.exp(z)
  return e / jnp.sum(e, axis=-1, keepdims=True)


def _tc_heads(n0, n1, lA, lB, lb, tA, tB, tb1, tW2, tb2, mA, mB, mb1, mW2,
              mb2):
  def body(n0_ref, n1_ref, lA_r, lB_r, lb_r, tA_r, tB_r, tb1_r, tW2_r, tb2_r,
           mA_r, mB_r, mb1_r, mW2_r, mb2_r, link_ref, type_ref, mod_ref):
    a = n0_ref[...]
    b = n1_ref[...]
    dot = lambda u, w: jnp.dot(u, w[...], preferred_element_type=jnp.float32)
    link_ref[...] = _softmax(dot(a, lA_r) + dot(b, lB_r) + lb_r[...])
    th = jax.nn.relu(dot(a, tA_r) + dot(b, tB_r) + tb1_r[...])
    type_ref[...] = _softmax(dot(th, tW2_r) + tb2_r[...])
    mh = jax.nn.relu(dot(a, mA_r) + dot(b, mB_r) + mb1_r[...])
    mod_ref[...] = _softmax(dot(mh, mW2_r) + mb2_r[...])

  full = lambda r, c: pl.BlockSpec((r, c), lambda i: (0, 0))
  return pl.pallas_call(
      body,
      grid=(PPAD // _PB,),
      in_specs=[
          pl.BlockSpec((_PB, 16), lambda i: (i, 0)),
          pl.BlockSpec((_PB, 16), lambda i: (i, 0)),
          full(16, 2), full(16, 2), full(1, 2),
          full(16, 16), full(16, 16), full(1, 16), full(16, 2), full(1, 2),
          full(16, 16), full(16, 16), full(1, 16), full(16, 4), full(1, 4),
      ],
      out_specs=[
          pl.BlockSpec((_PB, 2), lambda i: (i, 0)),
          pl.BlockSpec((_PB, 2), lambda i: (i, 0)),
          pl.BlockSpec((_PB, 4), lambda i: (i, 0)),
      ],
      out_shape=[_f32(PPAD, 2), _f32(PPAD, 2), _f32(PPAD, 4)],
  )(n0, n1, lA, lB, lb, tA, tB, tb1, tW2, tb2, mA, mB, mb1, mW2, mb2)


@jax.jit
def kernel(x, edge_index, edges, conv1_W, conv1_b, conv2_W, conv2_b,
           bn_gamma, bn_beta, bn_mean, bn_var, link_W, link_b, type_W1,
           type_b1, type_W2, type_b2, mod_W1, mod_b1, mod_W2, mod_b2):
  src = jnp.concatenate(
      [edge_index[0], jnp.zeros((EPAD - N_EDGES,), jnp.int32)])
  dst = jnp.concatenate(
      [edge_index[1],
       jnp.full((EPAD - N_EDGES,), N_NODES, jnp.int32)])
  src2d = src.reshape(EROWS, CH)
  dst2d = dst.reshape(EROWS, CH)
  ppad = jnp.zeros((PPAD - N_PAIRS,), jnp.int32)
  idx0 = jnp.concatenate([edges[:, 0], ppad]).reshape(PROWS, CH)
  idx1 = jnp.concatenate([edges[:, 1], ppad]).reshape(PROWS, CH)
  xpad = jnp.pad(x, ((0, ACC - N_NODES), (0, 0)))

  zeros8 = jnp.zeros((ZR, 8), jnp.float32)
  ones8 = jnp.ones((CH, 8), jnp.float32)

  bn_s = bn_gamma / jnp.sqrt(bn_var + 1e-5)
  W2f = conv2_W * bn_s[None, :]
  b2f = ((conv2_b - bn_mean) * bn_s + bn_beta).reshape(1, 16)

  deg8 = _sc_deg(dst2d, zeros8, ones8)
  xs = _tc_xs(xpad, deg8)
  (a1,) = _sc_segsum(src2d, dst2d, (xs,), zeros8)
  gl, gr = _tc_g(a1, xs, deg8, conv1_W, conv1_b.reshape(1, 32), W2f)
  a2l, a2r = _sc_segsum(src2d, dst2d, (gl, gr), zeros8)
  h2 = _tc_h2(a2l, a2r, gl, gr, deg8, b2f)
  n0, n1 = _sc_pair_gather(idx0, idx1, h2)
  link, typ, mod = _tc_heads(
      n0, n1, link_W[:16], link_W[16:], link_b.reshape(1, 2),
      type_W1[:16], type_W1[16:], type_b1.reshape(1, 16), type_W2,
      type_b2.reshape(1, 2), mod_W1[:16], mod_W1[16:], mod_b1.reshape(1, 16),
      mod_W2, mod_b2.reshape(1, 4))
  return link[:N_PAIRS], typ[:N_PAIRS], mod[:N_PAIRS]
